# Initial kernel scaffold; baseline (speedup 1.0000x reference)
import functools

import jax
import jax.numpy as jnp
from jax import lax
from jax.experimental import pallas as pl
from jax.experimental.pallas import tpu as pltpu
from jax.experimental.pallas import tpu_sc as plsc

_N = 10000
_D = 128
_P = 200000

_L = 16
_BC = 128

_ROWS = 400


def _xw_body(x_ref, w_ref, o_ref):
    o_ref[...] = jnp.dot(x_ref[...], w_ref[...],
                         preferred_element_type=jnp.float32)


def _gcn_body(adj_ref, xw_ref, h_ref, nrm_ref):
    h = jnp.dot(adj_ref[...], xw_ref[...],
                preferred_element_type=jnp.float32)
    h = jnp.maximum(h, 0.0)
    h_ref[...] = h
    nrm_ref[...] = jnp.sqrt(jnp.sum(h * h, axis=1, keepdims=True))


def _gcn(x, adj, W):
    xw = pl.pallas_call(
        _xw_body,
        out_shape=jax.ShapeDtypeStruct((_N, _D), jnp.float32),
    )(x, W)
    h, nrm = pl.pallas_call(
        _gcn_body,
        grid=(_N // _ROWS,),
        in_specs=[
            pl.BlockSpec((_ROWS, _N), lambda i: (i, 0)),
            pl.BlockSpec((_N, _D), lambda i: (0, 0)),
        ],
        out_specs=[
            pl.BlockSpec((_ROWS, _D), lambda i: (i, 0)),
            pl.BlockSpec((_ROWS, 1), lambda i: (i, 0)),
        ],
        out_shape=[
            jax.ShapeDtypeStruct((_N, _D), jnp.float32),
            jax.ShapeDtypeStruct((_N, 1), jnp.float32),
        ],
    )(adj, xw)
    return h, nrm


def _sc_cosine(h, nrm, n1p, n2p, p_pad):
    info = plsc.get_sparse_core_info()
    nw = info.num_cores * info.num_subcores
    per_w = p_pad // nw
    n_chunks = per_w // _BC
    mesh = plsc.VectorSubcoreMesh(core_axis_name="c", subcore_axis_name="s")

    @functools.partial(
        pl.kernel,
        mesh=mesh,
        out_type=jax.ShapeDtypeStruct((p_pad,), jnp.float32),
        scratch_types=[
            pltpu.VMEM((_N,), jnp.float32),
            pltpu.VMEM((_BC,), jnp.int32),
            pltpu.VMEM((_BC,), jnp.int32),
            pltpu.VMEM((_BC, _D), jnp.float32),
            pltpu.VMEM((_BC, _D), jnp.float32),
            pltpu.VMEM((_L, _L), jnp.float32),
            pltpu.VMEM((_BC,), jnp.float32),
            pltpu.SemaphoreType.DMA,
            pltpu.SemaphoreType.DMA,
        ],
    )
    def k(h_hbm, nrm_hbm, i1_hbm, i2_hbm, out_hbm,
          nrm_v, i1_v, i2_v, r1_v, r2_v, m_v, o_v, sem1, sem2):
        wid = lax.axis_index("s") * info.num_cores + lax.axis_index("c")
        base = wid * per_w
        pltpu.sync_copy(nrm_hbm, nrm_v)

        lanes = lax.iota(jnp.int32, _L)

        def chunk_body(c, carry):
            off = pl.multiple_of(base + c * _BC, 8)
            pltpu.sync_copy(i1_hbm.at[pl.ds(off, _BC)], i1_v)
            pltpu.sync_copy(i2_hbm.at[pl.ds(off, _BC)], i2_v)
            cp1 = pltpu.async_copy(h_hbm.at[i1_v], r1_v, sem1)
            cp2 = pltpu.async_copy(h_hbm.at[i2_v], r2_v, sem2)
            cp1.wait()
            cp2.wait()

            def group_body(g, carry2):
                gb = pl.multiple_of(g * _L, 8)
                i1 = i1_v[pl.ds(gb, _L)]
                i2 = i2_v[pl.ds(gb, _L)]
                nrm1 = plsc.load_gather(nrm_v, [i1])
                nrm2 = plsc.load_gather(nrm_v, [i2])
                for j in range(_L):
                    p = gb + j
                    acc = r1_v[p, pl.ds(0, _L)] * r2_v[p, pl.ds(0, _L)]
                    for t in range(1, _D // _L):
                        acc = acc + (r1_v[p, pl.ds(t * _L, _L)]
                                     * r2_v[p, pl.ds(t * _L, _L)])
                    plsc.store_scatter(
                        m_v, [lanes, jnp.full((_L,), j, jnp.int32)], acc)
                dots = m_v[0, :]
                for l in range(1, _L):
                    dots = dots + m_v[l, :]
                denom = jnp.maximum(nrm1 * nrm2, 1e-6)
                o_v[pl.ds(gb, _L)] = dots / denom
                return carry2

            lax.fori_loop(0, _BC // _L, group_body, 0)
            pltpu.sync_copy(o_v, out_hbm.at[pl.ds(off, _BC)])
            return carry

        lax.fori_loop(0, n_chunks, chunk_body, 0)

    return k(h, nrm, n1p, n2p)


def kernel(x, adj, node1, node2, W):
    h, nrm = _gcn(x, adj, W)
    info = plsc.get_sparse_core_info()
    nw = info.num_cores * info.num_subcores
    quantum = nw * _BC
    p_pad = ((_P + quantum - 1) // quantum) * quantum
    pad = p_pad - _P
    n1p = jnp.concatenate([node1, jnp.zeros((pad,), jnp.int32)])
    n2p = jnp.concatenate([node2, jnp.zeros((pad,), jnp.int32)])
    cos = _sc_cosine(h, nrm.reshape(_N), n1p, n2p, p_pad)
    return cos[:_P].reshape(_P, 1)


# speedup vs baseline: 2.4027x; 2.4027x over previous
import functools

import jax
import jax.numpy as jnp
from jax import lax
from jax.experimental import pallas as pl
from jax.experimental.pallas import tpu as pltpu
from jax.experimental.pallas import tpu_sc as plsc

_N = 10000
_D = 128
_P = 200000

_L = 16
_BC = 128

_ROWS = 400


def _xw_body(x_ref, w_ref, o_ref):
    o_ref[...] = jnp.dot(x_ref[...], w_ref[...],
                         preferred_element_type=jnp.float32)


def _gcn_body(adj_ref, xw_ref, h_ref, nrm_ref):
    h = jnp.dot(adj_ref[...], xw_ref[...],
                preferred_element_type=jnp.float32)
    h = jnp.maximum(h, 0.0)
    h_ref[...] = h
    nrm_ref[...] = jnp.sqrt(jnp.sum(h * h, axis=1, keepdims=True))


def _gcn(x, adj, W):
    xw = pl.pallas_call(
        _xw_body,
        out_shape=jax.ShapeDtypeStruct((_N, _D), jnp.float32),
    )(x, W)
    h, nrm = pl.pallas_call(
        _gcn_body,
        grid=(_N // _ROWS,),
        in_specs=[
            pl.BlockSpec((_ROWS, _N), lambda i: (i, 0)),
            pl.BlockSpec((_N, _D), lambda i: (0, 0)),
        ],
        out_specs=[
            pl.BlockSpec((_ROWS, _D), lambda i: (i, 0)),
            pl.BlockSpec((_ROWS, 1), lambda i: (i, 0)),
        ],
        out_shape=[
            jax.ShapeDtypeStruct((_N, _D), jnp.float32),
            jax.ShapeDtypeStruct((_N, 1), jnp.float32),
        ],
    )(adj, xw)
    return h, nrm


def _sc_cosine(h, nrm, n1p, n2p, p_pad):
    info = plsc.get_sparse_core_info()
    nw = info.num_cores * info.num_subcores
    per_w = p_pad // nw
    n_chunks = per_w // _BC
    mesh = plsc.VectorSubcoreMesh(core_axis_name="c", subcore_axis_name="s")

    @functools.partial(
        pl.kernel,
        mesh=mesh,
        out_type=jax.ShapeDtypeStruct((p_pad,), jnp.float32),
        compiler_params=pltpu.CompilerParams(needs_layout_passes=False),
        scratch_types=[
            pltpu.VMEM((_N,), jnp.float32),
            pltpu.VMEM((_BC,), jnp.int32),
            pltpu.VMEM((_BC,), jnp.int32),
            pltpu.VMEM((_BC, _D), jnp.float32),
            pltpu.VMEM((_BC, _D), jnp.float32),
            pltpu.VMEM((_L, _L), jnp.float32),
            pltpu.VMEM((_BC,), jnp.float32),
            pltpu.SemaphoreType.DMA,
            pltpu.SemaphoreType.DMA,
        ],
    )
    def k(h_hbm, nrm_hbm, i1_hbm, i2_hbm, out_hbm,
          nrm_v, i1_v, i2_v, r1_v, r2_v, m_v, o_v, sem1, sem2):
        wid = lax.axis_index("s") * info.num_cores + lax.axis_index("c")
        base = wid * per_w
        pltpu.sync_copy(nrm_hbm, nrm_v)

        lanes = lax.iota(jnp.int32, _L)

        def chunk_body(c, carry):
            off = pl.multiple_of(base + c * _BC, 8)
            pltpu.sync_copy(i1_hbm.at[pl.ds(off, _BC)], i1_v)
            pltpu.sync_copy(i2_hbm.at[pl.ds(off, _BC)], i2_v)
            cp1 = pltpu.async_copy(h_hbm.at[i1_v], r1_v, sem1)
            cp2 = pltpu.async_copy(h_hbm.at[i2_v], r2_v, sem2)
            cp1.wait()
            cp2.wait()

            def group_body(g, carry2):
                gb = pl.multiple_of(g * _L, 8)
                i1 = i1_v[pl.ds(gb, _L)]
                i2 = i2_v[pl.ds(gb, _L)]
                nrm1 = plsc.load_gather(nrm_v, [i1])
                nrm2 = plsc.load_gather(nrm_v, [i2])
                for j in range(_L):
                    p = gb + j
                    acc = r1_v[p, pl.ds(0, _L)] * r2_v[p, pl.ds(0, _L)]
                    for t in range(1, _D // _L):
                        acc = acc + (r1_v[p, pl.ds(t * _L, _L)]
                                     * r2_v[p, pl.ds(t * _L, _L)])
                    plsc.store_scatter(
                        m_v, [lanes, jnp.full((_L,), j, jnp.int32)], acc)
                dots = m_v[0, :]
                for l in range(1, _L):
                    dots = dots + m_v[l, :]
                denom = jnp.maximum(nrm1 * nrm2, 1e-6)
                o_v[pl.ds(gb, _L)] = dots / denom
                return carry2

            lax.fori_loop(0, _BC // _L, group_body, 0)
            pltpu.sync_copy(o_v, out_hbm.at[pl.ds(off, _BC)])
            return carry

        lax.fori_loop(0, n_chunks, chunk_body, 0)

    return k(h, nrm, n1p, n2p)


def kernel(x, adj, node1, node2, W):
    h, nrm = _gcn(x, adj, W)
    info = plsc.get_sparse_core_info()
    nw = info.num_cores * info.num_subcores
    quantum = nw * _BC
    p_pad = ((_P + quantum - 1) // quantum) * quantum
    pad = p_pad - _P
    n1p = jnp.concatenate([node1, jnp.zeros((pad,), jnp.int32)])
    n2p = jnp.concatenate([node2, jnp.zeros((pad,), jnp.int32)])
    cos = _sc_cosine(h, nrm.reshape(_N), n1p, n2p, p_pad)
    return cos[:_P].reshape(_P, 1)
